# baseline (speedup 1.0000x reference)
import jax
import jax.numpy as jnp
from jax import lax
from jax.experimental import pallas as pl
from jax.experimental.pallas import tpu as pltpu
from jax.experimental.pallas import tpu_sc as plsc

E = 16
D = 1024
T = 4096
TOKB = 1024
TM = 128
P = T + E * TM
NT = P // TM
NW = 32
TPW = T // NW
NBLK = T // TOKB


def _router_body(x_ref, rw_ref, rb_ref, xbf_ref, pos_ref, meta_ref,
                 e_s, carry_ref, tri_ref):
    j = pl.program_id(0)

    @pl.when(j == 0)
    def _init():
        carry_ref[...] = jnp.zeros_like(carry_ref)
        tr = lax.broadcasted_iota(jnp.int32, (TOKB, TOKB), 0)
        tc = lax.broadcasted_iota(jnp.int32, (TOKB, TOKB), 1)
        tri_ref[...] = (tr > tc).astype(jnp.float32)

    xb = x_ref[0]
    xbf = xb.astype(jnp.bfloat16)
    lo = lax.bitcast_convert_type(xbf[:, :D // 2], jnp.uint16).astype(jnp.uint32)
    hi = lax.bitcast_convert_type(xbf[:, D // 2:], jnp.uint16).astype(jnp.uint32)
    xbf_ref[...] = lax.bitcast_convert_type(lo | (hi << 16), jnp.int32)
    logits = jnp.dot(xb, rw_ref[...], preferred_element_type=jnp.float32)
    logits = logits + rb_ref[0:1, :]
    m = jnp.max(logits, axis=1, keepdims=True)
    lane = lax.broadcasted_iota(jnp.int32, (TOKB, E), 1)
    e_col = jnp.min(jnp.where(logits == m, lane, E), axis=1, keepdims=True)
    onehot = (lane == e_col).astype(jnp.float32)

    local_rank = jnp.dot(tri_ref[...], onehot,
                         preferred_element_type=jnp.float32)
    carry_row = carry_ref[0:1, 0:E]
    grank = jnp.sum((local_rank + carry_row) * onehot, axis=1, keepdims=True)
    new_carry = carry_row + jnp.sum(onehot, axis=0, keepdims=True)
    carry_ref[0:1, 0:E] = new_carry

    ps = lax.broadcasted_iota(jnp.int32, (8, TOKB), 0)
    pt = lax.broadcasted_iota(jnp.int32, (8, TOKB), 1)
    pk = ((pt // 128) == ps).astype(jnp.float32)
    ml = lax.broadcasted_iota(jnp.int32, (TOKB, 128), 0)
    ll = lax.broadcasted_iota(jnp.int32, (TOKB, 128), 1)
    bm = ((ml & 127) == ll).astype(jnp.float32)
    comb = e_col.astype(jnp.float32) * 4096.0 + grank
    c_packed = jnp.dot(pk, comb * bm, preferred_element_type=jnp.float32,
                       precision=lax.Precision.HIGHEST)
    rows = TOKB // 128
    e_s[pl.ds(rows * j, rows), :] = c_packed.astype(jnp.int32)

    @pl.when(j == NBLK - 1)
    def _plan():
        ci = new_carry.astype(jnp.int32)
        ri = ((ci + (TM - 1)) // TM) * TM
        rf = ri.astype(jnp.float32)
        packed = e_s[...]
        ea = packed >> 12
        pos = packed & 4095
        lane128 = lax.broadcasted_iota(jnp.int32, (1, 128), 1)
        jb = lane128 * TM
        te = jnp.full((1, 128), -1, jnp.int32)
        ts = jnp.zeros((1, 128), jnp.int32)
        nt = jnp.zeros((1, 128), jnp.int32)
        acc = jnp.zeros((1, 1), jnp.float32)
        for k in range(E):
            pk_off = acc.astype(jnp.int32)
            pos = pos + jnp.where(ea == k, pk_off, 0)
            te = te + jnp.where(jb >= pk_off, 1, 0)
            ts = ts + jnp.where(lane128 == k, pk_off // TM, 0)
            nt = nt + jnp.where(lane128 == k,
                                lax.slice(ri, (0, k), (1, k + 1)) // TM, 0)
            acc = acc + lax.slice(rf, (0, k), (1, k + 1))
        pos_ref[...] = pos
        meta_ref[0:1, :] = te
        meta_ref[1:2, :] = ts
        meta_ref[2:3, :] = nt


def _router_call(x_flat, rw, rb8):
    return pl.pallas_call(
        _router_body,
        grid=(NBLK,),
        in_specs=[
            pl.BlockSpec((1, TOKB, D), lambda j: (j // 2, j % 2, 0)),
            pl.BlockSpec((D, E), lambda j: (0, 0)),
            pl.BlockSpec((1, E), lambda j: (0, 0)),
        ],
        out_specs=[
            pl.BlockSpec((TOKB, D // 2), lambda j: (j, 0)),
            pl.BlockSpec((NW, 128), lambda j: (0, 0)),
            pl.BlockSpec((8, 128), lambda j: (0, 0)),
        ],
        out_shape=[
            jax.ShapeDtypeStruct((T, D // 2), jnp.int32),
            jax.ShapeDtypeStruct((NW, 128), jnp.int32),
            jax.ShapeDtypeStruct((8, 128), jnp.int32),
        ],
        scratch_shapes=[
            pltpu.VMEM((NW, 128), jnp.int32),
            pltpu.VMEM((8, 128), jnp.float32),
            pltpu.VMEM((TOKB, TOKB), jnp.float32),
        ],
        compiler_params=pltpu.CompilerParams(
            dimension_semantics=("arbitrary",)),
    )(x_flat, rw, rb8)


def _dispatch_body(x_hbm, pos_hbm, out_hbm, buf0, buf1, idx0, idx1, sem0,
                   sem1, sem2, sem3):
    wid = lax.axis_index("c") * 16 + lax.axis_index("s")
    base = wid * TPW
    pltpu.sync_copy(pos_hbm.at[wid, pl.ds(0, 64)], idx0)
    pltpu.sync_copy(pos_hbm.at[wid, pl.ds(64, 64)], idx1)
    l0 = pltpu.async_copy(x_hbm.at[pl.ds(base, 64)], buf0, sem0)
    l1 = pltpu.async_copy(x_hbm.at[pl.ds(base + 64, 64)], buf1, sem1)
    l0.wait()
    s0 = pltpu.async_copy(buf0, out_hbm.at[idx0], sem2)
    l1.wait()
    s1 = pltpu.async_copy(buf1, out_hbm.at[idx1], sem3)
    s0.wait()
    s1.wait()


def _dispatch_call(xbf, pos):
    f = pl.kernel(
        _dispatch_body,
        mesh=plsc.VectorSubcoreMesh(core_axis_name="c", subcore_axis_name="s"),
        out_type=jax.ShapeDtypeStruct((P, D // 2), jnp.int32),
        scratch_types=[
            pltpu.VMEM((64, D // 2), jnp.int32),
            pltpu.VMEM((64, D // 2), jnp.int32),
            pltpu.VMEM((64,), jnp.int32),
            pltpu.VMEM((64,), jnp.int32),
            pltpu.SemaphoreType.DMA,
            pltpu.SemaphoreType.DMA,
            pltpu.SemaphoreType.DMA,
            pltpu.SemaphoreType.DMA,
        ],
    )
    return f(xbf, pos)


def _combine_body(ff_hbm, pos_hbm, out_hbm, buf0, buf1, idx, sem0, sem1):
    wid = lax.axis_index("c") * 16 + lax.axis_index("s")
    b = wid // 16
    r0 = (wid % 16) * TPW
    pltpu.sync_copy(pos_hbm.at[wid], idx)
    g0 = pltpu.async_copy(ff_hbm.at[idx.at[pl.ds(0, 32)]], buf0, sem0)
    g1 = pltpu.async_copy(ff_hbm.at[idx.at[pl.ds(32, 32)]], buf1, sem1)
    g0.wait()
    pltpu.sync_copy(buf0, out_hbm.at[b, pl.ds(r0, 32)])
    g2 = pltpu.async_copy(ff_hbm.at[idx.at[pl.ds(64, 32)]], buf0, sem0)
    g1.wait()
    pltpu.sync_copy(buf1, out_hbm.at[b, pl.ds(r0 + 32, 32)])
    g3 = pltpu.async_copy(ff_hbm.at[idx.at[pl.ds(96, 32)]], buf1, sem1)
    g2.wait()
    pltpu.sync_copy(buf0, out_hbm.at[b, pl.ds(r0 + 64, 32)])
    g3.wait()
    pltpu.sync_copy(buf1, out_hbm.at[b, pl.ds(r0 + 96, 32)])


def _combine_call(ff, pos):
    f = pl.kernel(
        _combine_body,
        mesh=plsc.VectorSubcoreMesh(core_axis_name="c", subcore_axis_name="s"),
        out_type=jax.ShapeDtypeStruct((2, T // 2, D), jnp.float32),
        scratch_types=[
            pltpu.VMEM((32, D), jnp.float32),
            pltpu.VMEM((32, D), jnp.float32),
            pltpu.VMEM((TPW,), jnp.int32),
            pltpu.SemaphoreType.DMA,
            pltpu.SemaphoreType.DMA,
        ],
    )
    return f(ff, pos)


NRING = 4


def _ffn_body(meta_ref, x_hbm, wg_hbm, wu_hbm, wd_hbm, o_hbm,
              wg_r, wu_r, wd_r, xr, ostg, lsem, xsem, osem):
    def start_load(e, slot):
        pltpu.make_async_copy(wg_hbm.at[e], wg_r.at[slot], lsem.at[slot, 0]).start()
        pltpu.make_async_copy(wu_hbm.at[e], wu_r.at[slot], lsem.at[slot, 1]).start()
        pltpu.make_async_copy(wd_hbm.at[e], wd_r.at[slot], lsem.at[slot, 2]).start()

    def wait_load(e, slot):
        pltpu.make_async_copy(wg_hbm.at[e], wg_r.at[slot], lsem.at[slot, 0]).wait()
        pltpu.make_async_copy(wu_hbm.at[e], wu_r.at[slot], lsem.at[slot, 1]).wait()
        pltpu.make_async_copy(wd_hbm.at[e], wd_r.at[slot], lsem.at[slot, 2]).wait()

    def start_x(g2, slot):
        pltpu.make_async_copy(x_hbm.at[pl.ds(g2 * TM, TM)], xr.at[slot],
                              xsem.at[slot]).start()

    def wait_x(slot):
        pltpu.make_async_copy(x_hbm.at[pl.ds(0, TM)], xr.at[slot],
                              xsem.at[slot]).wait()

    for s in range(3):
        start_load(s, s)
    for s in range(4):
        start_x(s, s)

    def expert_body(e, g):
        slot = lax.rem(e, NRING)
        @pl.when(e + 3 < E)
        def _next():
            start_load(e + 3, lax.rem(e + 3, NRING))
        wait_load(e, slot)
        ts = meta_ref[1, e]
        ntl = meta_ref[2, e]

        def tile_body(i, g2):
            row = (ts + i) * TM
            pp = lax.rem(g2, 2)

            @pl.when(g2 >= 2)
            def _drain():
                pltpu.make_async_copy(ostg.at[pp], o_hbm.at[pl.ds(0, TM)],
                                      osem.at[pp]).wait()

            xslot = lax.rem(g2, 4)
            wait_x(xslot)
            xi = lax.bitcast_convert_type(xr[xslot], jnp.uint32)
            lo = lax.bitcast_convert_type((xi & 0xFFFF).astype(jnp.uint16),
                                          jnp.bfloat16).astype(jnp.float32)
            hi = lax.bitcast_convert_type((xi >> 16).astype(jnp.uint16),
                                          jnp.bfloat16).astype(jnp.float32)
            xb = jnp.concatenate([lo, hi], axis=1)
            gg = jnp.dot(xb, wg_r[slot], preferred_element_type=jnp.float32)
            uu = jnp.dot(xb, wu_r[slot], preferred_element_type=jnp.float32)
            a = gg * jax.nn.sigmoid(gg) * uu
            ostg[pp] = jnp.dot(a, wd_r[slot], preferred_element_type=jnp.float32)
            pltpu.make_async_copy(ostg.at[pp], o_hbm.at[pl.ds(row, TM)],
                                  osem.at[pp]).start()

            @pl.when(g2 + 4 < NT)
            def _nextx():
                start_x(g2 + 4, xslot)

            return g2 + 1

        g = lax.fori_loop(0, ntl, tile_body, g)
        return g

    g = lax.fori_loop(0, E, expert_body, 0)
    ndrain = jnp.minimum(4, NT - g)
    lax.fori_loop(0, ndrain,
                  lambda i, c: (wait_x(lax.rem(g + i, 4)), c)[1], 0)
    pltpu.make_async_copy(ostg.at[lax.rem(g + 1, 2)], o_hbm.at[pl.ds(0, TM)],
                          osem.at[lax.rem(g + 1, 2)]).wait()
    pltpu.make_async_copy(ostg.at[lax.rem(g, 2)], o_hbm.at[pl.ds(0, TM)],
                          osem.at[lax.rem(g, 2)]).wait()


def _ffn_call(meta, xs, wg, wu, wd):
    grid_spec = pltpu.PrefetchScalarGridSpec(
        num_scalar_prefetch=1,
        grid=(1,),
        in_specs=[
            pl.BlockSpec(memory_space=pl.ANY),
            pl.BlockSpec(memory_space=pl.ANY),
            pl.BlockSpec(memory_space=pl.ANY),
            pl.BlockSpec(memory_space=pl.ANY),
        ],
        out_specs=pl.BlockSpec(memory_space=pl.ANY),
        scratch_shapes=[
            pltpu.VMEM((NRING, D, D), jnp.float32),
            pltpu.VMEM((NRING, D, D), jnp.float32),
            pltpu.VMEM((NRING, D, D), jnp.float32),
            pltpu.VMEM((4, TM, D // 2), jnp.int32),
            pltpu.VMEM((2, TM, D), jnp.float32),
            pltpu.SemaphoreType.DMA((NRING, 3)),
            pltpu.SemaphoreType.DMA((4,)),
            pltpu.SemaphoreType.DMA((2,)),
        ],
    )
    return pl.pallas_call(
        _ffn_body,
        grid_spec=grid_spec,
        out_shape=jax.ShapeDtypeStruct((P, D), jnp.float32),
        compiler_params=pltpu.CompilerParams(
            dimension_semantics=("arbitrary",)),
    )(meta, xs, wg, wu, wd)


def kernel(x, router_w, router_b, we_gate, we_up, we_down):
    xsh = x.shape
    xbf, pos, meta = _router_call(x, router_w, router_b.reshape(1, E))
    xs = _dispatch_call(xbf, pos)
    ff = _ffn_call(meta, xs, we_gate, we_up, we_down)
    return _combine_call(ff, pos).reshape(xsh)


# speedup vs baseline: 1.0259x; 1.0259x over previous
import jax
import jax.numpy as jnp
from jax import lax
from jax.experimental import pallas as pl
from jax.experimental.pallas import tpu as pltpu
from jax.experimental.pallas import tpu_sc as plsc

E = 16
D = 1024
T = 4096
TOKB = 1024
TM = 128
P = T + E * TM
NT = P // TM
NW = 32
TPW = T // NW
NBLK = T // TOKB


def _router_body(x_ref, rw_ref, rb_ref, xbf_ref, pos_ref, meta_ref,
                 e_s, carry_ref, tri_ref):
    j = pl.program_id(0)

    @pl.when(j == 0)
    def _init():
        carry_ref[...] = jnp.zeros_like(carry_ref)
        tr = lax.broadcasted_iota(jnp.int32, (TOKB, TOKB), 0)
        tc = lax.broadcasted_iota(jnp.int32, (TOKB, TOKB), 1)
        tri_ref[...] = (tr > tc).astype(jnp.float32)

    xb = x_ref[0]
    xbf = xb.astype(jnp.bfloat16)
    lo = lax.bitcast_convert_type(xbf[:, :D // 2], jnp.uint16).astype(jnp.uint32)
    hi = lax.bitcast_convert_type(xbf[:, D // 2:], jnp.uint16).astype(jnp.uint32)
    xbf_ref[...] = lax.bitcast_convert_type(lo | (hi << 16), jnp.int32)
    logits = jnp.dot(xb, rw_ref[...], preferred_element_type=jnp.float32)
    logits = logits + rb_ref[0:1, :]
    m = jnp.max(logits, axis=1, keepdims=True)
    lane = lax.broadcasted_iota(jnp.int32, (TOKB, E), 1)
    e_col = jnp.min(jnp.where(logits == m, lane, E), axis=1, keepdims=True)
    onehot = (lane == e_col).astype(jnp.float32)

    local_rank = jnp.dot(tri_ref[...], onehot,
                         preferred_element_type=jnp.float32)
    carry_row = carry_ref[0:1, 0:E]
    grank = jnp.sum((local_rank + carry_row) * onehot, axis=1, keepdims=True)
    new_carry = carry_row + jnp.sum(onehot, axis=0, keepdims=True)
    carry_ref[0:1, 0:E] = new_carry

    ps = lax.broadcasted_iota(jnp.int32, (8, TOKB), 0)
    pt = lax.broadcasted_iota(jnp.int32, (8, TOKB), 1)
    pk = ((pt // 128) == ps).astype(jnp.float32)
    ml = lax.broadcasted_iota(jnp.int32, (TOKB, 128), 0)
    ll = lax.broadcasted_iota(jnp.int32, (TOKB, 128), 1)
    bm = ((ml & 127) == ll).astype(jnp.float32)
    comb = e_col.astype(jnp.float32) * 4096.0 + grank
    c_packed = jnp.dot(pk, comb * bm, preferred_element_type=jnp.float32,
                       precision=lax.Precision.HIGHEST)
    rows = TOKB // 128
    e_s[pl.ds(rows * j, rows), :] = c_packed.astype(jnp.int32)

    @pl.when(j == NBLK - 1)
    def _plan():
        ci = new_carry.astype(jnp.int32)
        ri = ((ci + (TM - 1)) // TM) * TM
        rf = ri.astype(jnp.float32)
        packed = e_s[...]
        ea = packed >> 12
        pos = packed & 4095
        lane128 = lax.broadcasted_iota(jnp.int32, (1, 128), 1)
        jb = lane128 * TM
        te = jnp.full((1, 128), -1, jnp.int32)
        ts = jnp.zeros((1, 128), jnp.int32)
        nt = jnp.zeros((1, 128), jnp.int32)
        acc = jnp.zeros((1, 1), jnp.float32)
        for k in range(E):
            pk_off = acc.astype(jnp.int32)
            pos = pos + jnp.where(ea == k, pk_off, 0)
            te = te + jnp.where(jb >= pk_off, 1, 0)
            ts = ts + jnp.where(lane128 == k, pk_off // TM, 0)
            nt = nt + jnp.where(lane128 == k,
                                lax.slice(ri, (0, k), (1, k + 1)) // TM, 0)
            acc = acc + lax.slice(rf, (0, k), (1, k + 1))
        pos_ref[...] = pos
        meta_ref[0:1, :] = te
        meta_ref[1:2, :] = ts
        meta_ref[2:3, :] = nt


def _router_call(x_flat, rw, rb8):
    return pl.pallas_call(
        _router_body,
        grid=(NBLK,),
        in_specs=[
            pl.BlockSpec((1, TOKB, D), lambda j: (j // 2, j % 2, 0)),
            pl.BlockSpec((D, E), lambda j: (0, 0)),
            pl.BlockSpec((1, E), lambda j: (0, 0)),
        ],
        out_specs=[
            pl.BlockSpec((TOKB, D // 2), lambda j: (j, 0)),
            pl.BlockSpec((NW, 128), lambda j: (0, 0)),
            pl.BlockSpec((8, 128), lambda j: (0, 0)),
        ],
        out_shape=[
            jax.ShapeDtypeStruct((T, D // 2), jnp.int32),
            jax.ShapeDtypeStruct((NW, 128), jnp.int32),
            jax.ShapeDtypeStruct((8, 128), jnp.int32),
        ],
        scratch_shapes=[
            pltpu.VMEM((NW, 128), jnp.int32),
            pltpu.VMEM((8, 128), jnp.float32),
            pltpu.VMEM((TOKB, TOKB), jnp.float32),
        ],
        compiler_params=pltpu.CompilerParams(
            dimension_semantics=("arbitrary",)),
    )(x_flat, rw, rb8)


def _dispatch_body(x_hbm, pos_hbm, out_hbm, buf0, buf1, idx0, idx1, sem0,
                   sem1, sem2, sem3):
    wid = lax.axis_index("c") * 16 + lax.axis_index("s")
    base = wid * TPW
    pltpu.sync_copy(pos_hbm.at[wid, pl.ds(0, 64)], idx0)
    pltpu.sync_copy(pos_hbm.at[wid, pl.ds(64, 64)], idx1)
    l0 = pltpu.async_copy(x_hbm.at[pl.ds(base, 64)], buf0, sem0)
    l1 = pltpu.async_copy(x_hbm.at[pl.ds(base + 64, 64)], buf1, sem1)
    l0.wait()
    s0 = pltpu.async_copy(buf0, out_hbm.at[idx0], sem2)
    l1.wait()
    s1 = pltpu.async_copy(buf1, out_hbm.at[idx1], sem3)
    s0.wait()
    s1.wait()


def _dispatch_call(xbf, pos):
    f = pl.kernel(
        _dispatch_body,
        mesh=plsc.VectorSubcoreMesh(core_axis_name="c", subcore_axis_name="s"),
        out_type=jax.ShapeDtypeStruct((P, D // 2), jnp.int32),
        scratch_types=[
            pltpu.VMEM((64, D // 2), jnp.int32),
            pltpu.VMEM((64, D // 2), jnp.int32),
            pltpu.VMEM((64,), jnp.int32),
            pltpu.VMEM((64,), jnp.int32),
            pltpu.SemaphoreType.DMA,
            pltpu.SemaphoreType.DMA,
            pltpu.SemaphoreType.DMA,
            pltpu.SemaphoreType.DMA,
        ],
    )
    return f(xbf, pos)


def _combine_body(ff_hbm, pos_hbm, out_hbm, buf0, buf1, idx, sem0, sem1):
    wid = lax.axis_index("c") * 16 + lax.axis_index("s")
    b = wid // 16
    r0 = (wid % 16) * TPW
    pltpu.sync_copy(pos_hbm.at[wid], idx)
    g0 = pltpu.async_copy(ff_hbm.at[idx.at[pl.ds(0, 32)]], buf0, sem0)
    g1 = pltpu.async_copy(ff_hbm.at[idx.at[pl.ds(32, 32)]], buf1, sem1)
    g0.wait()
    pltpu.sync_copy(buf0, out_hbm.at[b, pl.ds(r0, 32)])
    g2 = pltpu.async_copy(ff_hbm.at[idx.at[pl.ds(64, 32)]], buf0, sem0)
    g1.wait()
    pltpu.sync_copy(buf1, out_hbm.at[b, pl.ds(r0 + 32, 32)])
    g3 = pltpu.async_copy(ff_hbm.at[idx.at[pl.ds(96, 32)]], buf1, sem1)
    g2.wait()
    pltpu.sync_copy(buf0, out_hbm.at[b, pl.ds(r0 + 64, 32)])
    g3.wait()
    pltpu.sync_copy(buf1, out_hbm.at[b, pl.ds(r0 + 96, 32)])


def _combine_call(ff, pos):
    f = pl.kernel(
        _combine_body,
        mesh=plsc.VectorSubcoreMesh(core_axis_name="c", subcore_axis_name="s"),
        out_type=jax.ShapeDtypeStruct((2, T // 2, D), jnp.float32),
        scratch_types=[
            pltpu.VMEM((32, D), jnp.float32),
            pltpu.VMEM((32, D), jnp.float32),
            pltpu.VMEM((TPW,), jnp.int32),
            pltpu.SemaphoreType.DMA,
            pltpu.SemaphoreType.DMA,
        ],
    )
    return f(ff, pos)


NRING = 3


def _ffn_body(meta_ref, x_hbm, wg_hbm, wu_hbm, wd_hbm, o_hbm,
              wg_r, wu_r, wd_r, xr, ostg, lsem, xsem, osem):
    def start_load(e, slot):
        pltpu.make_async_copy(wg_hbm.at[e], wg_r.at[slot], lsem.at[slot, 0]).start()
        pltpu.make_async_copy(wu_hbm.at[e], wu_r.at[slot], lsem.at[slot, 1]).start()
        pltpu.make_async_copy(wd_hbm.at[e], wd_r.at[slot], lsem.at[slot, 2]).start()

    def wait_load(e, slot):
        pltpu.make_async_copy(wg_hbm.at[e], wg_r.at[slot], lsem.at[slot, 0]).wait()
        pltpu.make_async_copy(wu_hbm.at[e], wu_r.at[slot], lsem.at[slot, 1]).wait()
        pltpu.make_async_copy(wd_hbm.at[e], wd_r.at[slot], lsem.at[slot, 2]).wait()

    def start_x(g2, slot):
        pltpu.make_async_copy(x_hbm.at[pl.ds(g2 * TM, TM)], xr.at[slot],
                              xsem.at[slot]).start()

    def wait_x(slot):
        pltpu.make_async_copy(x_hbm.at[pl.ds(0, TM)], xr.at[slot],
                              xsem.at[slot]).wait()

    for s in range(2):
        start_load(s, s)
    for s in range(4):
        start_x(s, s)

    def expert_body(e, g):
        slot = lax.rem(e, NRING)
        @pl.when(e + 2 < E)
        def _next():
            start_load(e + 2, lax.rem(e + 2, NRING))
        wait_load(e, slot)
        ts = meta_ref[1, e]
        ntl = meta_ref[2, e]

        def tile_body(i, g2):
            row = (ts + i) * TM
            pp = lax.rem(g2, 2)

            @pl.when(g2 >= 2)
            def _drain():
                pltpu.make_async_copy(ostg.at[pp], o_hbm.at[pl.ds(0, TM)],
                                      osem.at[pp]).wait()

            xslot = lax.rem(g2, 4)
            wait_x(xslot)
            xi = lax.bitcast_convert_type(xr[xslot], jnp.uint32)
            lo = lax.bitcast_convert_type((xi & 0xFFFF).astype(jnp.uint16),
                                          jnp.bfloat16).astype(jnp.float32)
            hi = lax.bitcast_convert_type((xi >> 16).astype(jnp.uint16),
                                          jnp.bfloat16).astype(jnp.float32)
            xb = jnp.concatenate([lo, hi], axis=1)
            gg = jnp.dot(xb, wg_r[slot], preferred_element_type=jnp.float32)
            uu = jnp.dot(xb, wu_r[slot], preferred_element_type=jnp.float32)
            a = gg * jax.nn.sigmoid(gg) * uu
            ostg[pp] = jnp.dot(a, wd_r[slot], preferred_element_type=jnp.float32)
            pltpu.make_async_copy(ostg.at[pp], o_hbm.at[pl.ds(row, TM)],
                                  osem.at[pp]).start()

            @pl.when(g2 + 4 < NT)
            def _nextx():
                start_x(g2 + 4, xslot)

            return g2 + 1

        g = lax.fori_loop(0, ntl, tile_body, g)
        return g

    g = lax.fori_loop(0, E, expert_body, 0)
    ndrain = jnp.minimum(4, NT - g)
    lax.fori_loop(0, ndrain,
                  lambda i, c: (wait_x(lax.rem(g + i, 4)), c)[1], 0)
    pltpu.make_async_copy(ostg.at[lax.rem(g + 1, 2)], o_hbm.at[pl.ds(0, TM)],
                          osem.at[lax.rem(g + 1, 2)]).wait()
    pltpu.make_async_copy(ostg.at[lax.rem(g, 2)], o_hbm.at[pl.ds(0, TM)],
                          osem.at[lax.rem(g, 2)]).wait()


def _ffn_call(meta, xs, wg, wu, wd):
    grid_spec = pltpu.PrefetchScalarGridSpec(
        num_scalar_prefetch=1,
        grid=(1,),
        in_specs=[
            pl.BlockSpec(memory_space=pl.ANY),
            pl.BlockSpec(memory_space=pl.ANY),
            pl.BlockSpec(memory_space=pl.ANY),
            pl.BlockSpec(memory_space=pl.ANY),
        ],
        out_specs=pl.BlockSpec(memory_space=pl.ANY),
        scratch_shapes=[
            pltpu.VMEM((NRING, D, D), jnp.float32),
            pltpu.VMEM((NRING, D, D), jnp.float32),
            pltpu.VMEM((NRING, D, D), jnp.float32),
            pltpu.VMEM((4, TM, D // 2), jnp.int32),
            pltpu.VMEM((2, TM, D), jnp.float32),
            pltpu.SemaphoreType.DMA((NRING, 3)),
            pltpu.SemaphoreType.DMA((4,)),
            pltpu.SemaphoreType.DMA((2,)),
        ],
    )
    return pl.pallas_call(
        _ffn_body,
        grid_spec=grid_spec,
        out_shape=jax.ShapeDtypeStruct((P, D), jnp.float32),
        compiler_params=pltpu.CompilerParams(
            dimension_semantics=("arbitrary",)),
    )(meta, xs, wg, wu, wd)


def kernel(x, router_w, router_b, we_gate, we_up, we_down):
    xsh = x.shape
    xbf, pos, meta = _router_call(x, router_w, router_b.reshape(1, E))
    xs = _dispatch_call(xbf, pos)
    ff = _ffn_call(meta, xs, we_gate, we_up, we_down)
    return _combine_call(ff, pos).reshape(xsh)
